# baseline (speedup 1.0000x reference)
import functools

import jax
import jax.numpy as jnp
from jax import lax
from jax.experimental import pallas as pl
from jax.experimental.pallas import tpu as pltpu
from jax.experimental.pallas import tpu_sc as plsc

N_NODES = 10000
N_EDGES = 320000
H = 128

NC = 2
NS = 16
NW = NC * NS

EPW = N_EDGES // NW
CHUNK = 80
NCH = EPW // CHUNK
NBUF = 3
NTAIL = NCH % NBUF
NMAIN = NCH - NTAIL
N_PAD = 10240
ROWS_PER_TILE = N_PAD // NS

def _segment_sum_body(dst_hbm, edges_hbm, zeros_hbm, out_hbm, idx_v, ebuf, acc, *sems):
    c = lax.axis_index("c")
    s = lax.axis_index("s")
    wid = c * NS + s
    base = wid * EPW

    pltpu.sync_copy(dst_hbm.at[wid], idx_v)

    pltpu.sync_copy(
        zeros_hbm.at[pl.ds(s * ROWS_PER_TILE, ROWS_PER_TILE)],
        acc.at[pl.ds(s * ROWS_PER_TILE, ROWS_PER_TILE)],
    )
    plsc.subcore_barrier()

    gsems = sems[:NBUF]
    ssems = sems[NBUF:]

    for b in range(NBUF):
        pltpu.make_async_copy(
            edges_hbm.at[pl.ds(base + b * CHUNK, CHUNK)], ebuf.at[b], gsems[b]
        ).start()

    @pl.loop(0, NMAIN, step=NBUF)
    def _(j0):
        for b in range(NBUF):
            j = j0 + b
            pltpu.make_async_copy(
                edges_hbm.at[pl.ds(base + j * CHUNK, CHUNK)], ebuf.at[b], gsems[b]
            ).wait()
            pltpu.make_async_copy(
                ebuf.at[b], acc.at[idx_v.at[j]], ssems[b]
            ).start(add=True)
        for b in range(NBUF):
            j = j0 + b

            @pl.when(j + NBUF < NCH)
            def _():
                pltpu.make_async_copy(
                    ebuf.at[b], acc.at[idx_v.at[j]], ssems[b]
                ).wait()
                pltpu.make_async_copy(
                    edges_hbm.at[pl.ds(base + (j + NBUF) * CHUNK, CHUNK)],
                    ebuf.at[b],
                    gsems[b],
                ).start()

    for b in range(NBUF):
        j = NMAIN - NBUF + b
        if j + NBUF >= NCH:
            pltpu.make_async_copy(
                ebuf.at[b], acc.at[idx_v.at[j]], ssems[b]
            ).wait()

    for t in range(NTAIL):
        j = NMAIN + t
        b = j % NBUF
        pltpu.make_async_copy(
            edges_hbm.at[pl.ds(base + j * CHUNK, CHUNK)], ebuf.at[b], gsems[b]
        ).wait()
        pltpu.sync_copy(ebuf.at[b], acc.at[idx_v.at[j]], add=True)

    plsc.subcore_barrier()
    pltpu.sync_copy(
        acc.at[pl.ds(s * ROWS_PER_TILE, ROWS_PER_TILE)],
        out_hbm.at[c].at[pl.ds(s * ROWS_PER_TILE, ROWS_PER_TILE)],
    )


@functools.lru_cache(maxsize=1)
def _segment_sum_sc():
    mesh = plsc.VectorSubcoreMesh(
        core_axis_name="c", subcore_axis_name="s", num_cores=NC, num_subcores=NS
    )
    return pl.kernel(
        _segment_sum_body,
        out_type=jax.ShapeDtypeStruct((NC, N_PAD, H), jnp.float32),
        mesh=mesh,
        scratch_types=[
            pltpu.VMEM((NCH, CHUNK), jnp.int32),
            pltpu.VMEM((NBUF, CHUNK, H), jnp.float32),
            pltpu.VMEM_SHARED((N_PAD, H), jnp.float32),
        ] + [pltpu.SemaphoreType.DMA] * (2 * NBUF),
    )


BLK = 2000


def _mlp_body(x_ref, p_ref, g_ref, w1_ref, b1_ref, w2_ref, b2_ref, gm_ref, bt_ref, o_ref):
    x = x_ref[...]
    agg = p_ref[0] + p_ref[1]
    w1 = w1_ref[...]
    h = (
        jnp.dot(x, w1[:H], preferred_element_type=jnp.float32)
        + jnp.dot(agg, w1[H:2 * H], preferred_element_type=jnp.float32)
        + jnp.dot(g_ref[...], w1[2 * H:], preferred_element_type=jnp.float32)
        + b1_ref[...]
    )
    h = jnp.maximum(h, 0.0)
    h = jnp.dot(h, w2_ref[...], preferred_element_type=jnp.float32) + b2_ref[...]
    mu = jnp.mean(h, axis=-1, keepdims=True)
    d = h - mu
    var = jnp.mean(d * d, axis=-1, keepdims=True)
    o_ref[...] = d * lax.rsqrt(var + 1e-5) * gm_ref[...] + bt_ref[...]


@jax.jit
def _mlp_tc(x, partials, global_attr, W1, b1, W2, b2, ln_gamma, ln_beta):
    grid = (N_NODES // BLK,)
    return pl.pallas_call(
        _mlp_body,
        grid=grid,
        in_specs=[
            pl.BlockSpec((BLK, H), lambda i: (i, 0)),
            pl.BlockSpec((NC, BLK, H), lambda i: (0, i, 0)),
            pl.BlockSpec((1, H), lambda i: (0, 0)),
            pl.BlockSpec((3 * H, H), lambda i: (0, 0)),
            pl.BlockSpec((1, H), lambda i: (0, 0)),
            pl.BlockSpec((H, H), lambda i: (0, 0)),
            pl.BlockSpec((1, H), lambda i: (0, 0)),
            pl.BlockSpec((1, H), lambda i: (0, 0)),
            pl.BlockSpec((1, H), lambda i: (0, 0)),
        ],
        out_specs=pl.BlockSpec((BLK, H), lambda i: (i, 0)),
        out_shape=jax.ShapeDtypeStruct((N_NODES, H), jnp.float32),
    )(x, partials, global_attr, W1, b1.reshape(1, H), W2, b2.reshape(1, H),
      ln_gamma.reshape(1, H), ln_beta.reshape(1, H))


def kernel(x, edge_index, edge_attr_updated, global_attr, W1, b1, W2, b2, ln_gamma, ln_beta):
    dst = edge_index[1].astype(jnp.int32).reshape(NW, NCH, CHUNK)
    zeros = jnp.zeros((N_PAD, H), jnp.float32)
    partials = _segment_sum_sc()(dst, edge_attr_updated, zeros)
    return _mlp_tc(x, partials, global_attr, W1, b1, W2, b2, ln_gamma, ln_beta)


# speedup vs baseline: 1.0320x; 1.0320x over previous
import functools

import jax
import jax.numpy as jnp
from jax import lax
from jax.experimental import pallas as pl
from jax.experimental.pallas import tpu as pltpu
from jax.experimental.pallas import tpu_sc as plsc

N_NODES = 10000
N_EDGES = 320000
H = 128

NC = 2
NS = 16
NW = NC * NS

EPW = N_EDGES // NW
CHUNK = 80
NCH = EPW // CHUNK
NBUF = 2
NTAIL = NCH % NBUF
NMAIN = NCH - NTAIL
N_PAD = 10240
ROWS_PER_TILE = N_PAD // NS

def _segment_sum_body(dst_hbm, edges_hbm, zeros_hbm, out_hbm, idx_v, ebuf, acc, *sems):
    c = lax.axis_index("c")
    s = lax.axis_index("s")
    wid = c * NS + s
    base = wid * EPW

    pltpu.sync_copy(dst_hbm.at[wid], idx_v)

    pltpu.sync_copy(
        zeros_hbm,
        acc.at[pl.ds(s * ROWS_PER_TILE, ROWS_PER_TILE)],
    )
    plsc.subcore_barrier()

    for b in range(NBUF):
        pltpu.make_async_copy(
            edges_hbm.at[pl.ds(base + b * CHUNK, CHUNK)], ebuf.at[b], sems[b]
        ).start()

    @pl.loop(0, NMAIN, step=NBUF)
    def _(j0):
        for b in range(NBUF):
            j = j0 + b
            pltpu.make_async_copy(
                edges_hbm.at[pl.ds(base + j * CHUNK, CHUNK)], ebuf.at[b], sems[b]
            ).wait()
            pltpu.sync_copy(ebuf.at[b], acc.at[idx_v.at[j]], add=True)

            @pl.when(j + NBUF < NCH)
            def _():
                pltpu.make_async_copy(
                    edges_hbm.at[pl.ds(base + (j + NBUF) * CHUNK, CHUNK)],
                    ebuf.at[b],
                    sems[b],
                ).start()

    for t in range(NTAIL):
        j = NMAIN + t
        b = j % NBUF
        pltpu.make_async_copy(
            edges_hbm.at[pl.ds(base + j * CHUNK, CHUNK)], ebuf.at[b], sems[b]
        ).wait()
        pltpu.sync_copy(ebuf.at[b], acc.at[idx_v.at[j]], add=True)

    plsc.subcore_barrier()
    pltpu.sync_copy(
        acc.at[pl.ds(s * ROWS_PER_TILE, ROWS_PER_TILE)],
        out_hbm.at[c].at[pl.ds(s * ROWS_PER_TILE, ROWS_PER_TILE)],
    )


@functools.lru_cache(maxsize=1)
def _segment_sum_sc():
    mesh = plsc.VectorSubcoreMesh(
        core_axis_name="c", subcore_axis_name="s", num_cores=NC, num_subcores=NS
    )
    return pl.kernel(
        _segment_sum_body,
        out_type=jax.ShapeDtypeStruct((NC, N_PAD, H), jnp.float32),
        mesh=mesh,
        scratch_types=[
            pltpu.VMEM((NCH, CHUNK), jnp.int32),
            pltpu.VMEM((NBUF, CHUNK, H), jnp.float32),
            pltpu.VMEM_SHARED((N_PAD, H), jnp.float32),
        ] + [pltpu.SemaphoreType.DMA] * NBUF,
    )


BLK = 2000


def _mlp_body(x_ref, p_ref, g_ref, w1_ref, b1_ref, w2_ref, b2_ref, gm_ref, bt_ref, o_ref):
    x = x_ref[...]
    agg = p_ref[0] + p_ref[1]
    w1 = w1_ref[...]
    h = (
        jnp.dot(x, w1[:H], preferred_element_type=jnp.float32)
        + jnp.dot(agg, w1[H:2 * H], preferred_element_type=jnp.float32)
        + jnp.dot(g_ref[...], w1[2 * H:], preferred_element_type=jnp.float32)
        + b1_ref[...]
    )
    h = jnp.maximum(h, 0.0)
    h = jnp.dot(h, w2_ref[...], preferred_element_type=jnp.float32) + b2_ref[...]
    mu = jnp.mean(h, axis=-1, keepdims=True)
    d = h - mu
    var = jnp.mean(d * d, axis=-1, keepdims=True)
    o_ref[...] = d * lax.rsqrt(var + 1e-5) * gm_ref[...] + bt_ref[...]


@jax.jit
def _mlp_tc(x, partials, global_attr, W1, b1, W2, b2, ln_gamma, ln_beta):
    grid = (N_NODES // BLK,)
    return pl.pallas_call(
        _mlp_body,
        grid=grid,
        in_specs=[
            pl.BlockSpec((BLK, H), lambda i: (i, 0)),
            pl.BlockSpec((NC, BLK, H), lambda i: (0, i, 0)),
            pl.BlockSpec((1, H), lambda i: (0, 0)),
            pl.BlockSpec((3 * H, H), lambda i: (0, 0)),
            pl.BlockSpec((1, H), lambda i: (0, 0)),
            pl.BlockSpec((H, H), lambda i: (0, 0)),
            pl.BlockSpec((1, H), lambda i: (0, 0)),
            pl.BlockSpec((1, H), lambda i: (0, 0)),
            pl.BlockSpec((1, H), lambda i: (0, 0)),
        ],
        out_specs=pl.BlockSpec((BLK, H), lambda i: (i, 0)),
        out_shape=jax.ShapeDtypeStruct((N_NODES, H), jnp.float32),
    )(x, partials, global_attr, W1, b1.reshape(1, H), W2, b2.reshape(1, H),
      ln_gamma.reshape(1, H), ln_beta.reshape(1, H))


def kernel(x, edge_index, edge_attr_updated, global_attr, W1, b1, W2, b2, ln_gamma, ln_beta):
    dst = edge_index[1].astype(jnp.int32).reshape(NW, NCH, CHUNK)
    zeros = jnp.zeros((ROWS_PER_TILE, H), jnp.float32)
    partials = _segment_sum_sc()(dst, edge_attr_updated, zeros)
    return _mlp_tc(x, partials, global_attr, W1, b1, W2, b2, ln_gamma, ln_beta)


# speedup vs baseline: 1.0480x; 1.0155x over previous
import functools

import jax
import jax.numpy as jnp
from jax import lax
from jax.experimental import pallas as pl
from jax.experimental.pallas import tpu as pltpu
from jax.experimental.pallas import tpu_sc as plsc

N_NODES = 10000
N_EDGES = 320000
H = 128

NC = 2
NS = 16
NW = NC * NS

EPW = N_EDGES // NW
CHUNK = 80
NCH = EPW // CHUNK
NBUF = 2
NTAIL = NCH % NBUF
NMAIN = NCH - NTAIL
N_PAD = 10240
ROWS_PER_TILE = N_PAD // NS

def _segment_sum_body(dst_hbm, edges_hbm, zeros_hbm, out_hbm, idxr, ebuf, acc, *sems):
    c = lax.axis_index("c")
    s = lax.axis_index("s")
    wid = c * NS + s
    base = wid * EPW
    esems = sems[:NBUF]
    isems = sems[NBUF:]

    def start_fetch(j, b):
        pltpu.make_async_copy(
            edges_hbm.at[pl.ds(base + j * CHUNK, CHUNK)], ebuf.at[b], esems[b]
        ).start()
        pltpu.make_async_copy(
            dst_hbm.at[pl.ds(base + j * CHUNK, CHUNK)], idxr.at[b], isems[b]
        ).start()

    def wait_fetch(j, b):
        pltpu.make_async_copy(
            edges_hbm.at[pl.ds(base + j * CHUNK, CHUNK)], ebuf.at[b], esems[b]
        ).wait()
        pltpu.make_async_copy(
            dst_hbm.at[pl.ds(base + j * CHUNK, CHUNK)], idxr.at[b], isems[b]
        ).wait()

    pltpu.sync_copy(
        zeros_hbm,
        acc.at[pl.ds(s * ROWS_PER_TILE, ROWS_PER_TILE)],
    )
    plsc.subcore_barrier()

    for b in range(NBUF):
        start_fetch(b, b)

    @pl.loop(0, NMAIN, step=NBUF)
    def _(j0):
        for b in range(NBUF):
            j = j0 + b
            wait_fetch(j, b)
            pltpu.sync_copy(ebuf.at[b], acc.at[idxr.at[b]], add=True)

            @pl.when(j + NBUF < NCH)
            def _():
                start_fetch(j + NBUF, b)

    for t in range(NTAIL):
        j = NMAIN + t
        b = j % NBUF
        wait_fetch(j, b)
        pltpu.sync_copy(ebuf.at[b], acc.at[idxr.at[b]], add=True)

    plsc.subcore_barrier()
    pltpu.sync_copy(
        acc.at[pl.ds(s * ROWS_PER_TILE, ROWS_PER_TILE)],
        out_hbm.at[c].at[pl.ds(s * ROWS_PER_TILE, ROWS_PER_TILE)],
    )


@functools.lru_cache(maxsize=1)
def _segment_sum_sc():
    mesh = plsc.VectorSubcoreMesh(
        core_axis_name="c", subcore_axis_name="s", num_cores=NC, num_subcores=NS
    )
    return pl.kernel(
        _segment_sum_body,
        out_type=jax.ShapeDtypeStruct((NC, N_PAD, H), jnp.float32),
        mesh=mesh,
        scratch_types=[
            pltpu.VMEM((NBUF, CHUNK), jnp.int32),
            pltpu.VMEM((NBUF, CHUNK, H), jnp.float32),
            pltpu.VMEM_SHARED((N_PAD, H), jnp.float32),
        ] + [pltpu.SemaphoreType.DMA] * (2 * NBUF),
    )


BLK = 2000


def _mlp_body(x_ref, p_ref, g_ref, w1_ref, b1_ref, w2_ref, b2_ref, gm_ref, bt_ref, o_ref):
    x = x_ref[...]
    agg = p_ref[0] + p_ref[1]
    w1 = w1_ref[...]
    h = (
        jnp.dot(x, w1[:H], preferred_element_type=jnp.float32)
        + jnp.dot(agg, w1[H:2 * H], preferred_element_type=jnp.float32)
        + jnp.dot(g_ref[...], w1[2 * H:], preferred_element_type=jnp.float32)
        + b1_ref[...]
    )
    h = jnp.maximum(h, 0.0)
    h = jnp.dot(h, w2_ref[...], preferred_element_type=jnp.float32) + b2_ref[...]
    mu = jnp.mean(h, axis=-1, keepdims=True)
    d = h - mu
    var = jnp.mean(d * d, axis=-1, keepdims=True)
    o_ref[...] = d * lax.rsqrt(var + 1e-5) * gm_ref[...] + bt_ref[...]


@jax.jit
def _mlp_tc(x, partials, global_attr, W1, b1, W2, b2, ln_gamma, ln_beta):
    grid = (N_NODES // BLK,)
    return pl.pallas_call(
        _mlp_body,
        grid=grid,
        in_specs=[
            pl.BlockSpec((BLK, H), lambda i: (i, 0)),
            pl.BlockSpec((NC, BLK, H), lambda i: (0, i, 0)),
            pl.BlockSpec((1, H), lambda i: (0, 0)),
            pl.BlockSpec((3 * H, H), lambda i: (0, 0)),
            pl.BlockSpec((1, H), lambda i: (0, 0)),
            pl.BlockSpec((H, H), lambda i: (0, 0)),
            pl.BlockSpec((1, H), lambda i: (0, 0)),
            pl.BlockSpec((1, H), lambda i: (0, 0)),
            pl.BlockSpec((1, H), lambda i: (0, 0)),
        ],
        out_specs=pl.BlockSpec((BLK, H), lambda i: (i, 0)),
        out_shape=jax.ShapeDtypeStruct((N_NODES, H), jnp.float32),
    )(x, partials, global_attr, W1, b1.reshape(1, H), W2, b2.reshape(1, H),
      ln_gamma.reshape(1, H), ln_beta.reshape(1, H))


def kernel(x, edge_index, edge_attr_updated, global_attr, W1, b1, W2, b2, ln_gamma, ln_beta):
    dst = edge_index[1].astype(jnp.int32)
    zeros = jnp.zeros((ROWS_PER_TILE, H), jnp.float32)
    partials = _segment_sum_sc()(dst, edge_attr_updated, zeros)
    return _mlp_tc(x, partials, global_attr, W1, b1, W2, b2, ln_gamma, ln_beta)


# speedup vs baseline: 1.2442x; 1.1872x over previous
import functools

import jax
import jax.numpy as jnp
from jax import lax
from jax.experimental import pallas as pl
from jax.experimental.pallas import tpu as pltpu
from jax.experimental.pallas import tpu_sc as plsc

N_NODES = 10000
N_EDGES = 320000
H = 128

NC = 2
NS = 16
NW = NC * NS

CHUNK = 128
NCHT = N_EDGES // CHUNK
KMAX = -(-NCHT // NW)
NREM = NCHT % NW
NBUF = 2
KPAD = -(-KMAX // NBUF) * NBUF
N_PAD = 10240
ROWS_PER_TILE = N_PAD // NS

def _segment_sum_body(ei_hbm, edges_hbm, zeros_hbm, out_hbm, idxr, ebuf, acc, *sems):
    c = lax.axis_index("c")
    s = lax.axis_index("s")
    wid = c * NS + s
    kw = jnp.where(wid < NREM, KMAX, KMAX - 1)
    esems = sems[:NBUF]
    isems = sems[NBUF:]

    def start_fetch(k, b):
        ch = wid + k * NW
        pltpu.make_async_copy(
            edges_hbm.at[pl.ds(ch * CHUNK, CHUNK)], ebuf.at[b], esems[b]
        ).start()
        pltpu.make_async_copy(
            ei_hbm.at[:, pl.ds(ch * CHUNK, CHUNK)], idxr.at[b], isems[b]
        ).start()

    def wait_fetch(k, b):
        ch = wid + k * NW
        pltpu.make_async_copy(
            edges_hbm.at[pl.ds(ch * CHUNK, CHUNK)], ebuf.at[b], esems[b]
        ).wait()
        pltpu.make_async_copy(
            ei_hbm.at[:, pl.ds(ch * CHUNK, CHUNK)], idxr.at[b], isems[b]
        ).wait()

    pltpu.sync_copy(
        zeros_hbm,
        acc.at[pl.ds(s * ROWS_PER_TILE, ROWS_PER_TILE)],
    )
    plsc.subcore_barrier()

    for b in range(NBUF):

        @pl.when(b < kw)
        def _():
            start_fetch(b, b)

    @pl.loop(0, KPAD, step=NBUF)
    def _(k0):
        for b in range(NBUF):
            k = k0 + b

            @pl.when(k < kw)
            def _():
                wait_fetch(k, b)
                pltpu.sync_copy(ebuf.at[b], acc.at[idxr.at[b].at[1]], add=True)

                @pl.when(k + NBUF < kw)
                def _():
                    start_fetch(k + NBUF, b)

    plsc.subcore_barrier()
    pltpu.sync_copy(
        acc.at[pl.ds(s * ROWS_PER_TILE, ROWS_PER_TILE)],
        out_hbm.at[c].at[pl.ds(s * ROWS_PER_TILE, ROWS_PER_TILE)],
    )


@functools.lru_cache(maxsize=1)
def _segment_sum_sc():
    mesh = plsc.VectorSubcoreMesh(
        core_axis_name="c", subcore_axis_name="s", num_cores=NC, num_subcores=NS
    )
    return pl.kernel(
        _segment_sum_body,
        out_type=jax.ShapeDtypeStruct((NC, N_PAD, H), jnp.float32),
        mesh=mesh,
        scratch_types=[
            pltpu.VMEM((NBUF, 2, CHUNK), jnp.int32),
            pltpu.VMEM((NBUF, CHUNK, H), jnp.float32),
            pltpu.VMEM_SHARED((N_PAD, H), jnp.float32),
        ] + [pltpu.SemaphoreType.DMA] * (2 * NBUF),
    )


BLK = 2000


def _mlp_body(x_ref, p_ref, g_ref, w1_ref, b1_ref, w2_ref, b2_ref, gm_ref, bt_ref, o_ref):
    x = x_ref[...]
    agg = p_ref[0] + p_ref[1]
    w1 = w1_ref[...]
    h = (
        jnp.dot(x, w1[:H], preferred_element_type=jnp.float32)
        + jnp.dot(agg, w1[H:2 * H], preferred_element_type=jnp.float32)
        + jnp.dot(g_ref[...], w1[2 * H:], preferred_element_type=jnp.float32)
        + b1_ref[...]
    )
    h = jnp.maximum(h, 0.0)
    h = jnp.dot(h, w2_ref[...], preferred_element_type=jnp.float32) + b2_ref[...]
    mu = jnp.mean(h, axis=-1, keepdims=True)
    d = h - mu
    var = jnp.mean(d * d, axis=-1, keepdims=True)
    o_ref[...] = d * lax.rsqrt(var + 1e-5) * gm_ref[...] + bt_ref[...]


@jax.jit
def _mlp_tc(x, partials, global_attr, W1, b1, W2, b2, ln_gamma, ln_beta):
    grid = (N_NODES // BLK,)
    return pl.pallas_call(
        _mlp_body,
        grid=grid,
        in_specs=[
            pl.BlockSpec((BLK, H), lambda i: (i, 0)),
            pl.BlockSpec((NC, BLK, H), lambda i: (0, i, 0)),
            pl.BlockSpec((1, H), lambda i: (0, 0)),
            pl.BlockSpec((3 * H, H), lambda i: (0, 0)),
            pl.BlockSpec((1, H), lambda i: (0, 0)),
            pl.BlockSpec((H, H), lambda i: (0, 0)),
            pl.BlockSpec((1, H), lambda i: (0, 0)),
            pl.BlockSpec((1, H), lambda i: (0, 0)),
            pl.BlockSpec((1, H), lambda i: (0, 0)),
        ],
        out_specs=pl.BlockSpec((BLK, H), lambda i: (i, 0)),
        out_shape=jax.ShapeDtypeStruct((N_NODES, H), jnp.float32),
    )(x, partials, global_attr, W1, b1.reshape(1, H), W2, b2.reshape(1, H),
      ln_gamma.reshape(1, H), ln_beta.reshape(1, H))


def kernel(x, edge_index, edge_attr_updated, global_attr, W1, b1, W2, b2, ln_gamma, ln_beta):
    ei = edge_index.astype(jnp.int32)
    zeros = jnp.zeros((ROWS_PER_TILE, H), jnp.float32)
    partials = _segment_sum_sc()(ei, edge_attr_updated, zeros)
    return _mlp_tc(x, partials, global_attr, W1, b1, W2, b2, ln_gamma, ln_beta)
